# baseline (speedup 1.0000x reference)
import functools

import jax
import jax.numpy as jnp
from jax import lax
from jax.experimental import pallas as pl
from jax.experimental.pallas import tpu as pltpu
from jax.experimental.pallas import tpu_sc as plsc

_HID = 128
_GAMMA = 12.0
_ERANGE = (12.0 + 2.0) / _HID
_PI = 3.141592653589793
_PHASE_SCALE = _PI / _ERANGE

_B = 4096
_NW = 32
_BPW = _B // _NW
_NSTAGE = 4
_STAGE = _BPW // _NSTAGE
_LANES = 16

_NREL = 1000
_RROW = 64
_RPASS = 32

_COS = (1.0, -0.5, 0.0416666641831398, -0.0013888862449675798,
        2.4800550818326883e-05, -2.7534767355064105e-07,
        2.060333015307947e-09, -9.72173383462227e-12)
_SIN = (1.0, -0.1666666716337204, 0.00833333283662796,
        -0.00019841254106722772, 2.75567026619683e-06,
        -2.5038682238687215e-08, 1.589647441457842e-10,
        -6.610122063425983e-13)


def _horner(coefs, y):
    acc = jnp.full((_LANES,), coefs[-1], jnp.float32)
    for c in coefs[-2::-1]:
        acc = acc * y + c
    return acc


def _sc_score(ent_hbm, rel_hbm, sample_hbm, out_hbm, cs_hbm,
              sv, iv, hv, tv, cv, rl, csl, pv, ov,
              sh0, sh1, st0, st1, sc0, sc1):
    cid = lax.axis_index("c")
    sid = lax.axis_index("s")
    wid = sid * 2 + cid
    base = wid * _BPW
    lane = lax.iota(jnp.int32, _LANES)
    pltpu.sync_copy(sample_hbm.at[pl.ds(base, _BPW)], sv)
    for g in range(_BPW // _LANES):
        rows = lane + (g * _LANES)
        for j in range(3):
            iv[j, pl.ds(g * _LANES, _LANES)] = plsc.load_gather(
                sv, [rows, jnp.full((_LANES,), j, jnp.int32)])
    hsems = (sh0, sh1)
    tsems = (st0, st1)
    csems = (sc0, sc1)

    def fire_ht(stage):
        par = stage % 2
        slc = pl.ds(stage * _STAGE, _STAGE)
        return (
            pltpu.async_copy(ent_hbm.at[iv.at[0, slc]], hv.at[par],
                             hsems[par]),
            pltpu.async_copy(ent_hbm.at[iv.at[2, slc]], tv.at[par],
                             tsems[par]),
        )

    ht_inflight = [fire_ht(0), fire_ht(1)]

    r0 = jnp.minimum(sid * _RROW, _NREL - _RROW)

    def cs_row(i, carry):
        for c in range(_HID // _LANES):
            lo = c * _LANES
            x = rl[i, pl.ds(lo, _LANES)] * _PHASE_SCALE
            y = x * x
            csl[i, pl.ds(lo, _LANES)] = _horner(_COS, y)
            csl[i, pl.ds(_HID + lo, _LANES)] = x * _horner(_SIN, y)
        return carry

    for p in range(_RROW // _RPASS):
        pltpu.sync_copy(rel_hbm.at[pl.ds(r0 + p * _RPASS, _RPASS)], rl)
        lax.fori_loop(0, _RPASS, cs_row, 0)
        pltpu.sync_copy(csl, cs_hbm.at[cid, pl.ds(r0 + p * _RPASS, _RPASS)])
    plsc.subcore_barrier()

    def fire_cs(stage):
        return pltpu.async_copy(
            cs_hbm.at[cid].at[iv.at[1, pl.ds(stage * _STAGE, _STAGE)]],
            cv.at[stage % 2], csems[stage % 2])

    def pair(par, stage, i):
        gi = stage * _STAGE + i
        acc = jnp.zeros((_LANES,), jnp.float32)
        for c in range(_HID // _LANES):
            lo = c * _LANES
            reh = hv[par, i, pl.ds(lo, _LANES)]
            imh = hv[par, i, pl.ds(_HID + lo, _LANES)]
            ret = tv[par, i, pl.ds(lo, _LANES)]
            imt = tv[par, i, pl.ds(_HID + lo, _LANES)]
            cr = cv[par, i, pl.ds(lo, _LANES)]
            sr = cv[par, i, pl.ds(_HID + lo, _LANES)]
            re = reh * cr - imh * sr - ret
            im = reh * sr + imh * cr - imt
            s = re * re + im * im
            bits = lax.bitcast_convert_type(s, jnp.int32)
            r = lax.bitcast_convert_type(
                jnp.int32(0x5F3759DF) - (bits >> 1), jnp.float32)
            sh = 0.5 * s
            r = r * (1.5 - sh * r * r)
            r = r * (1.5 - sh * r * r)
            acc = acc + s * r
        pv[gi, pl.ds(0, _LANES)] = acc

    inflight = fire_cs(0)
    for stage in range(_NSTAGE):
        nxt = fire_cs(stage + 1) if stage + 1 < _NSTAGE else None
        inflight.wait()
        for c in ht_inflight[stage]:
            c.wait()

        def body(i2, carry, par=stage % 2, stage=stage):
            pair(par, stage, i2 * 2)
            pair(par, stage, i2 * 2 + 1)
            return carry

        lax.fori_loop(0, _STAGE // 2, body, 0)
        if stage + 2 < _NSTAGE:
            ht_inflight.append(fire_ht(stage + 2))
        inflight = nxt

    for g in range(_BPW // _LANES):
        rows = lane + (g * _LANES)
        tot = jnp.zeros((_LANES,), jnp.float32)
        for j in range(_LANES):
            tot = tot + plsc.load_gather(pv, [rows, jnp.full((_LANES,), j,
                                                             jnp.int32)])
        ov[pl.ds(g * _LANES, _LANES)] = _GAMMA - tot
    pltpu.sync_copy(ov, out_hbm.at[pl.ds(base, _BPW)])


@functools.partial(
    pl.kernel,
    mesh=plsc.VectorSubcoreMesh(core_axis_name="c", subcore_axis_name="s"),
    compiler_params=pltpu.CompilerParams(needs_layout_passes=False),
    out_type=(
        jax.ShapeDtypeStruct((_B,), jnp.float32),
        jax.ShapeDtypeStruct((2, _NREL, 2 * _HID), jnp.float32),
    ),
    scratch_types=[
        pltpu.VMEM((_BPW, 3), jnp.int32),
        pltpu.VMEM((3, _BPW), jnp.int32),
        pltpu.VMEM((2, _STAGE, 2 * _HID), jnp.float32),
        pltpu.VMEM((2, _STAGE, 2 * _HID), jnp.float32),
        pltpu.VMEM((2, _STAGE, 2 * _HID), jnp.float32),
        pltpu.VMEM((_RPASS, _HID), jnp.float32),
        pltpu.VMEM((_RPASS, 2 * _HID), jnp.float32),
        pltpu.VMEM((_BPW, 17), jnp.float32),
        pltpu.VMEM((_BPW,), jnp.float32),
        pltpu.SemaphoreType.DMA,
        pltpu.SemaphoreType.DMA,
        pltpu.SemaphoreType.DMA,
        pltpu.SemaphoreType.DMA,
        pltpu.SemaphoreType.DMA,
        pltpu.SemaphoreType.DMA,
    ],
)
def _sc_kernel(ent_hbm, rel_hbm, sample_hbm, out_hbm, cs_hbm, *rest):
    _sc_score(ent_hbm, rel_hbm, sample_hbm, out_hbm, cs_hbm, *rest)


def kernel(sample, ent_emb, rel_emb):
    out, _ = _sc_kernel(ent_emb, rel_emb, sample.astype(jnp.int32))
    return out.reshape(_B, 1)


# speedup vs baseline: 1.4423x; 1.4423x over previous
import functools

import jax
import jax.numpy as jnp
from jax import lax
from jax.experimental import pallas as pl
from jax.experimental.pallas import tpu as pltpu
from jax.experimental.pallas import tpu_sc as plsc

_HID = 128
_GAMMA = 12.0
_ERANGE = (12.0 + 2.0) / _HID
_PI = 3.141592653589793
_PHASE_SCALE = _PI / _ERANGE

_B = 4096
_NW = 32
_BPW = _B // _NW
_NSTAGE = 4
_STAGE = _BPW // _NSTAGE
_LANES = 16


def _cs_body(rel_ref, cs_ref):
    ph = rel_ref[...] * _PHASE_SCALE
    cs_ref[:, :_HID] = jnp.cos(ph)
    cs_ref[:, _HID:] = jnp.sin(ph)


def _make_cs_table(rel_emb):
    n = rel_emb.shape[0]
    return pl.pallas_call(
        _cs_body,
        out_shape=jax.ShapeDtypeStruct((n, 2 * _HID), jnp.float32),
    )(rel_emb)


def _sc_score(ent_hbm, cs_hbm, hidx_hbm, ridx_hbm, tidx_hbm, out_hbm,
              iv, hv, tv, cv, pv, ov, sh0, sh1, st0, st1, sc0, sc1):
    wid = lax.axis_index("s") * 2 + lax.axis_index("c")
    base = wid * _BPW
    lane = lax.iota(jnp.int32, _LANES)
    pltpu.sync_copy(hidx_hbm.at[pl.ds(base, _BPW)], iv.at[0])
    pltpu.sync_copy(ridx_hbm.at[pl.ds(base, _BPW)], iv.at[1])
    pltpu.sync_copy(tidx_hbm.at[pl.ds(base, _BPW)], iv.at[2])
    hsems = (sh0, sh1)
    tsems = (st0, st1)
    csems = (sc0, sc1)

    def fire(stage):
        par = stage % 2
        slc = pl.ds(stage * _STAGE, _STAGE)
        return (
            pltpu.async_copy(ent_hbm.at[iv.at[0, slc]], hv.at[par],
                             hsems[par]),
            pltpu.async_copy(ent_hbm.at[iv.at[2, slc]], tv.at[par],
                             tsems[par]),
            pltpu.async_copy(cs_hbm.at[iv.at[1, slc]], cv.at[par],
                             csems[par]),
        )

    def pair(par, stage, i):
        acc = jnp.zeros((_LANES,), jnp.float32)
        for c in range(_HID // _LANES):
            lo = c * _LANES
            reh = hv[par, i, pl.ds(lo, _LANES)]
            imh = hv[par, i, pl.ds(_HID + lo, _LANES)]
            ret = tv[par, i, pl.ds(lo, _LANES)]
            imt = tv[par, i, pl.ds(_HID + lo, _LANES)]
            cr = cv[par, i, pl.ds(lo, _LANES)]
            sr = cv[par, i, pl.ds(_HID + lo, _LANES)]
            re = reh * cr - imh * sr - ret
            im = reh * sr + imh * cr - imt
            s = re * re + im * im
            bits = lax.bitcast_convert_type(s, jnp.int32)
            r = lax.bitcast_convert_type(
                jnp.int32(0x5F3759DF) - (bits >> 1), jnp.float32)
            sh = 0.5 * s
            r = r * (1.5 - sh * r * r)
            r = r * (1.5 - sh * r * r)
            acc = acc + s * r
        pv[i + stage * _STAGE, pl.ds(0, _LANES)] = acc

    inflight = fire(0)
    for stage in range(_NSTAGE):
        nxt = fire(stage + 1) if stage + 1 < _NSTAGE else None
        for c in inflight:
            c.wait()

        def body(i2, carry, par=stage % 2, stage=stage):
            pair(par, stage, i2 * 2)
            pair(par, stage, i2 * 2 + 1)
            return carry

        lax.fori_loop(0, _STAGE // 2, body, 0)
        inflight = nxt

    for g in range(_BPW // _LANES):
        rows = lane + (g * _LANES)
        tot = jnp.zeros((_LANES,), jnp.float32)
        for j in range(_LANES):
            tot = tot + plsc.load_gather(pv, [rows, jnp.full((_LANES,), j,
                                                             jnp.int32)])
        ov[pl.ds(g * _LANES, _LANES)] = _GAMMA - tot
    pltpu.sync_copy(ov, out_hbm.at[pl.ds(base, _BPW)])


@functools.partial(
    pl.kernel,
    mesh=plsc.VectorSubcoreMesh(core_axis_name="c", subcore_axis_name="s"),
    compiler_params=pltpu.CompilerParams(needs_layout_passes=False),
    out_type=jax.ShapeDtypeStruct((_B,), jnp.float32),
    scratch_types=[
        pltpu.VMEM((3, _BPW), jnp.int32),
        pltpu.VMEM((2, _STAGE, 2 * _HID), jnp.float32),
        pltpu.VMEM((2, _STAGE, 2 * _HID), jnp.float32),
        pltpu.VMEM((2, _STAGE, 2 * _HID), jnp.float32),
        pltpu.VMEM((_BPW, 17), jnp.float32),
        pltpu.VMEM((_BPW,), jnp.float32),
        pltpu.SemaphoreType.DMA,
        pltpu.SemaphoreType.DMA,
        pltpu.SemaphoreType.DMA,
        pltpu.SemaphoreType.DMA,
        pltpu.SemaphoreType.DMA,
        pltpu.SemaphoreType.DMA,
    ],
)
def _sc_kernel(ent_hbm, cs_hbm, hidx_hbm, ridx_hbm, tidx_hbm, out_hbm, *rest):
    _sc_score(ent_hbm, cs_hbm, hidx_hbm, ridx_hbm, tidx_hbm, out_hbm, *rest)


def kernel(sample, ent_emb, rel_emb):
    sample = sample.astype(jnp.int32)
    cs = _make_cs_table(rel_emb)
    out = _sc_kernel(ent_emb, cs, sample[:, 0], sample[:, 1], sample[:, 2])
    return out.reshape(_B, 1)
